# baseline (speedup 1.0000x reference)
import jax
import jax.numpy as jnp
from jax import lax
from jax.experimental import pallas as pl
from jax.experimental.pallas import tpu as pltpu
from jax.experimental.pallas import tpu_sc as plsc

UNITS = 128
CHUNK = 128


def _sc_body(table_hbm, idx_hbm, out_hbm, table_sp, idx_v, rows_v, sem):
    info = plsc.get_sparse_core_info()
    nc, ns = info.num_cores, info.num_subcores
    nw = nc * ns
    cid = lax.axis_index("c")
    sid = lax.axis_index("s")
    wid = sid * nc + cid

    @pl.when(sid == 0)
    def _():
        pltpu.sync_copy(table_hbm, table_sp)

    plsc.subcore_barrier()

    n_rows = out_hbm.shape[0]
    rows_per_w = n_rows // nw
    n_chunks = rows_per_w // CHUNK
    base = wid * rows_per_w

    def step(g, carry):
        off = base + g * CHUNK
        pltpu.sync_copy(idx_hbm.at[pl.ds(off, CHUNK)], idx_v)
        pltpu.async_copy(table_sp.at[idx_v], rows_v, sem).wait()
        pltpu.sync_copy(rows_v, out_hbm.at[pl.ds(off, CHUNK)])
        return carry

    lax.fori_loop(0, n_chunks, step, 0)


def kernel(inputs, w1, w2):
    B, T = inputs.shape
    U = w1.shape[1]
    idx = inputs.reshape(-1).astype(jnp.int32)
    table = jnp.concatenate([jnp.zeros_like(w1), w1, w2], axis=0)
    mesh = plsc.VectorSubcoreMesh(core_axis_name="c", subcore_axis_name="s")
    k = pl.kernel(
        _sc_body,
        out_type=jax.ShapeDtypeStruct((B * T, U), jnp.float32),
        mesh=mesh,
        scratch_types=[
            pltpu.VMEM_SHARED((3, U), jnp.float32),
            pltpu.VMEM((CHUNK,), jnp.int32),
            pltpu.VMEM((CHUNK, U), jnp.float32),
            pltpu.SemaphoreType.DMA,
        ],
    )
    out = k(table, idx)
    return out.reshape(B, T, U)


# speedup vs baseline: 1.9627x; 1.9627x over previous
import jax
import jax.numpy as jnp
from jax import lax
from jax.experimental import pallas as pl
from jax.experimental.pallas import tpu as pltpu
from jax.experimental.pallas import tpu_sc as plsc

UNITS = 128
CHUNK = 128


def _sc_body(table_hbm, idx_hbm, out_hbm, table_sp, idx_all, rows0, rows1,
             sem_in0, sem_in1, sem_out0, sem_out1):
    info = plsc.get_sparse_core_info()
    nc, ns = info.num_cores, info.num_subcores
    nw = nc * ns
    cid = lax.axis_index("c")
    sid = lax.axis_index("s")
    wid = sid * nc + cid

    @pl.when(sid == 0)
    def _():
        pltpu.sync_copy(table_hbm, table_sp)

    plsc.subcore_barrier()

    n_rows = out_hbm.shape[0]
    rows_per_w = n_rows // nw
    n_chunks = rows_per_w // CHUNK
    n_pairs = n_chunks // 2
    base = wid * rows_per_w

    pltpu.sync_copy(idx_hbm.at[pl.ds(base, rows_per_w)], idx_all)

    def idx_at(c):
        return idx_all.at[pl.ds(c * CHUNK, CHUNK)]

    def gather(c, rows, sem):
        return pltpu.async_copy(table_sp.at[idx_at(c)], rows, sem)

    def put(c, rows, sem):
        return pltpu.async_copy(rows, out_hbm.at[pl.ds(base + c * CHUNK, CHUNK)], sem)

    gather(0, rows0, sem_in0)

    def pair(g, carry):
        c0 = 2 * g
        pltpu.make_async_copy(table_sp.at[idx_at(c0)], rows0, sem_in0).wait()
        put(c0, rows0, sem_out0)

        @pl.when(g >= 1)
        def _():
            pltpu.make_async_copy(rows1, out_hbm.at[pl.ds(base, CHUNK)], sem_out1).wait()

        gather(c0 + 1, rows1, sem_in1)
        pltpu.make_async_copy(table_sp.at[idx_at(c0 + 1)], rows1, sem_in1).wait()
        put(c0 + 1, rows1, sem_out1)
        pltpu.make_async_copy(rows0, out_hbm.at[pl.ds(base, CHUNK)], sem_out0).wait()

        @pl.when(g + 1 < n_pairs)
        def _():
            gather(c0 + 2, rows0, sem_in0)

        return carry

    lax.fori_loop(0, n_pairs, pair, 0)
    pltpu.make_async_copy(rows1, out_hbm.at[pl.ds(base, CHUNK)], sem_out1).wait()


def kernel(inputs, w1, w2):
    B, T = inputs.shape
    U = w1.shape[1]
    idx = inputs.reshape(-1).astype(jnp.int32)
    table = jnp.concatenate([jnp.zeros_like(w1), w1, w2], axis=0)
    mesh = plsc.VectorSubcoreMesh(core_axis_name="c", subcore_axis_name="s")
    rows_per_w = (B * T) // 32
    k = pl.kernel(
        _sc_body,
        out_type=jax.ShapeDtypeStruct((B * T, U), jnp.float32),
        mesh=mesh,
        scratch_types=[
            pltpu.VMEM_SHARED((3, U), jnp.float32),
            pltpu.VMEM((rows_per_w,), jnp.int32),
            pltpu.VMEM((CHUNK, U), jnp.float32),
            pltpu.VMEM((CHUNK, U), jnp.float32),
            pltpu.SemaphoreType.DMA,
            pltpu.SemaphoreType.DMA,
            pltpu.SemaphoreType.DMA,
            pltpu.SemaphoreType.DMA,
        ],
    )
    out = k(table, idx)
    return out.reshape(B, T, U)


# speedup vs baseline: 1.9947x; 1.0163x over previous
import jax
import jax.numpy as jnp
from jax import lax
from jax.experimental import pallas as pl
from jax.experimental.pallas import tpu as pltpu
from jax.experimental.pallas import tpu_sc as plsc

UNITS = 128
CHUNK = 128
NBUF = 4
LOOKAHEAD = 2


def _sc_body(table_hbm, idx_hbm, out_hbm, table_sp, idx_all,
             rows0, rows1, rows2, rows3,
             sin0, sin1, sin2, sin3, sout0, sout1, sout2, sout3):
    rows = (rows0, rows1, rows2, rows3)
    sin = (sin0, sin1, sin2, sin3)
    sout = (sout0, sout1, sout2, sout3)

    info = plsc.get_sparse_core_info()
    nc, ns = info.num_cores, info.num_subcores
    nw = nc * ns
    cid = lax.axis_index("c")
    sid = lax.axis_index("s")
    wid = sid * nc + cid

    @pl.when(sid == 0)
    def _():
        pltpu.sync_copy(table_hbm, table_sp)

    plsc.subcore_barrier()

    n_rows = out_hbm.shape[0]
    rows_per_w = n_rows // nw
    n_chunks = rows_per_w // CHUNK
    n_groups = n_chunks // NBUF
    base = wid * rows_per_w

    pltpu.sync_copy(idx_hbm.at[pl.ds(base, rows_per_w)], idx_all)

    def gather(c, b):
        return pltpu.async_copy(table_sp.at[idx_all.at[pl.ds(c * CHUNK, CHUNK)]],
                                rows[b], sin[b])

    def wait_gather(b):
        pltpu.make_async_copy(table_sp.at[idx_all.at[pl.ds(0, CHUNK)]],
                              rows[b], sin[b]).wait()

    def put(c, b):
        return pltpu.async_copy(rows[b],
                                out_hbm.at[pl.ds(base + c * CHUNK, CHUNK)], sout[b])

    def wait_put(b):
        pltpu.make_async_copy(rows[b], out_hbm.at[pl.ds(base, CHUNK)], sout[b]).wait()

    for c in range(LOOKAHEAD):
        gather(c, c % NBUF)

    def group(g, carry):
        for db in range(NBUF):
            c = NBUF * g + db
            bg = (db + LOOKAHEAD) % NBUF

            @pl.when(jnp.logical_and(c + LOOKAHEAD < n_chunks,
                                     c + LOOKAHEAD >= NBUF))
            def _():
                wait_put(bg)

            @pl.when(c + LOOKAHEAD < n_chunks)
            def _():
                gather(c + LOOKAHEAD, bg)

            wait_gather(db)
            put(c, db)
        return carry

    lax.fori_loop(0, n_groups, group, 0)

    for b in range(NBUF):
        wait_put(b)


def kernel(inputs, w1, w2):
    B, T = inputs.shape
    U = w1.shape[1]
    idx = inputs.reshape(-1).astype(jnp.int32)
    table = jnp.concatenate([jnp.zeros_like(w1), w1, w2], axis=0)
    mesh = plsc.VectorSubcoreMesh(core_axis_name="c", subcore_axis_name="s")
    rows_per_w = (B * T) // 32
    k = pl.kernel(
        _sc_body,
        out_type=jax.ShapeDtypeStruct((B * T, U), jnp.float32),
        mesh=mesh,
        scratch_types=[
            pltpu.VMEM_SHARED((3, U), jnp.float32),
            pltpu.VMEM((rows_per_w,), jnp.int32),
            pltpu.VMEM((CHUNK, U), jnp.float32),
            pltpu.VMEM((CHUNK, U), jnp.float32),
            pltpu.VMEM((CHUNK, U), jnp.float32),
            pltpu.VMEM((CHUNK, U), jnp.float32),
            pltpu.SemaphoreType.DMA,
            pltpu.SemaphoreType.DMA,
            pltpu.SemaphoreType.DMA,
            pltpu.SemaphoreType.DMA,
            pltpu.SemaphoreType.DMA,
            pltpu.SemaphoreType.DMA,
            pltpu.SemaphoreType.DMA,
            pltpu.SemaphoreType.DMA,
        ],
    )
    out = k(table, idx)
    return out.reshape(B, T, U)
